# Initial kernel scaffold; baseline (speedup 1.0000x reference)
import jax
import jax.numpy as jnp
from jax.experimental import pallas as pl


def kernel(x, Wg, bg, W1, b1, W2, b2):
    raise NotImplementedError("write your pallas kernel here")


# speedup vs baseline: 1.0112x; 1.0112x over previous
import functools
import math

import jax
import jax.numpy as jnp
from jax import lax
from jax.experimental import pallas as pl
from jax.experimental.pallas import tpu as pltpu

_D = 1024
_E = 16
_H = 4096
_T = 128
_CAP = float(_T)
_EPS = 1e-6
_HT = 2048


def _gate_body(x_ref, wg_ref, bg_ref, gate_ref):
    logits = jnp.dot(x_ref[...], wg_ref[...],
                     preferred_element_type=jnp.float32) + bg_ref[...]
    m = jnp.max(logits, axis=1, keepdims=True)
    ex = jnp.exp(logits - m)
    p = ex / jnp.sum(ex, axis=1, keepdims=True)
    iota = lax.broadcasted_iota(jnp.int32, (_T, _E), 1)
    pm = jnp.max(p, axis=1, keepdims=True)
    ismax = p >= pm
    first = jnp.min(jnp.where(ismax, iota, _E), axis=1, keepdims=True)
    masked = jnp.where(iota == first, p, 0.0)
    denom = jnp.sum(masked, axis=0, keepdims=True) + _EPS
    gate_ref[...] = masked / denom * _CAP


def _ffn_body(x_ref, gate_ref, w1_ref, b1_ref, w2_ref, b2_ref, out_ref):
    e = pl.program_id(0)
    j = pl.program_id(1)

    @pl.when((e == 0) & (j == 0))
    def _init():
        out_ref[...] = jnp.zeros_like(out_ref)

    iota = lax.broadcasted_iota(jnp.int32, (_T, _E), 1)
    g = jnp.sum(jnp.where(iota == e, gate_ref[...], 0.0),
                axis=1, keepdims=True)
    h = jnp.dot(x_ref[...], w1_ref[0],
                preferred_element_type=jnp.float32) + b1_ref[0]
    h = 0.5 * h * (1.0 + lax.erf(h * (1.0 / math.sqrt(2.0))))
    out_ref[...] += jnp.dot(g * h, w2_ref[0],
                            preferred_element_type=jnp.float32)

    @pl.when(j == 0)
    def _bias2():
        out_ref[...] += g * b2_ref[0]


def kernel(x, Wg, bg, W1, b1, W2, b2):
    gate = pl.pallas_call(
        _gate_body,
        out_shape=jax.ShapeDtypeStruct((_T, _E), jnp.float32),
    )(x, Wg, bg.reshape(1, _E))

    nj = _H // _HT
    out = pl.pallas_call(
        _ffn_body,
        grid=(_E, nj),
        in_specs=[
            pl.BlockSpec((_T, _D), lambda e, j: (0, 0)),
            pl.BlockSpec((_T, _E), lambda e, j: (0, 0)),
            pl.BlockSpec((1, _D, _HT), lambda e, j: (e, 0, j)),
            pl.BlockSpec((1, 1, _HT), lambda e, j: (e, 0, j)),
            pl.BlockSpec((1, _HT, _D), lambda e, j: (e, j, 0)),
            pl.BlockSpec((1, 1, _D), lambda e, j: (e, 0, 0)),
        ],
        out_specs=pl.BlockSpec((_T, _D), lambda e, j: (0, 0)),
        out_shape=jax.ShapeDtypeStruct((_T, _D), jnp.float32),
        compiler_params=pltpu.CompilerParams(
            dimension_semantics=("arbitrary", "arbitrary"),
        ),
    )(x, gate, W1, b1.reshape(_E, 1, _H), W2, b2.reshape(_E, 1, _D))
    return out


# speedup vs baseline: 1.0220x; 1.0106x over previous
import functools
import math

import jax
import jax.numpy as jnp
from jax import lax
from jax.experimental import pallas as pl
from jax.experimental.pallas import tpu as pltpu

_D = 1024
_E = 16
_H = 4096
_T = 128
_CAP = float(_T)
_EPS = 1e-6
_HT = 2048


def _ffn_body(x_ref, wg_ref, bg_ref, w1_ref, b1_ref, w2_ref, b2_ref,
              out_ref, gate_ref):
    e = pl.program_id(0)
    j = pl.program_id(1)

    @pl.when((e == 0) & (j == 0))
    def _gate_and_init():
        logits = jnp.dot(x_ref[...], wg_ref[...],
                         preferred_element_type=jnp.float32) + bg_ref[...]
        m = jnp.max(logits, axis=1, keepdims=True)
        ex = jnp.exp(logits - m)
        p = ex / jnp.sum(ex, axis=1, keepdims=True)
        iota = lax.broadcasted_iota(jnp.int32, (_T, _E), 1)
        pm = jnp.max(p, axis=1, keepdims=True)
        first = jnp.min(jnp.where(p >= pm, iota, _E), axis=1, keepdims=True)
        masked = jnp.where(iota == first, p, 0.0)
        denom = jnp.sum(masked, axis=0, keepdims=True) + _EPS
        gate_ref[...] = masked / denom * _CAP
        out_ref[...] = jnp.zeros_like(out_ref)

    iota = lax.broadcasted_iota(jnp.int32, (_T, _E), 1)
    g = jnp.sum(jnp.where(iota == e, gate_ref[...], 0.0),
                axis=1, keepdims=True)
    h = jnp.dot(x_ref[...], w1_ref[0],
                preferred_element_type=jnp.float32) + b1_ref[0]
    h = 0.5 * h * (1.0 + lax.erf(h * (1.0 / math.sqrt(2.0))))
    out_ref[...] += jnp.dot(g * h, w2_ref[0],
                            preferred_element_type=jnp.float32)

    @pl.when(j == 0)
    def _bias2():
        out_ref[...] += g * b2_ref[0]


def kernel(x, Wg, bg, W1, b1, W2, b2):
    nj = _H // _HT
    out = pl.pallas_call(
        _ffn_body,
        grid=(_E, nj),
        in_specs=[
            pl.BlockSpec((_T, _D), lambda e, j: (0, 0)),
            pl.BlockSpec((_D, _E), lambda e, j: (0, 0)),
            pl.BlockSpec((1, _E), lambda e, j: (0, 0)),
            pl.BlockSpec((1, _D, _HT), lambda e, j: (e, 0, j)),
            pl.BlockSpec((1, 1, _HT), lambda e, j: (e, 0, j)),
            pl.BlockSpec((1, _HT, _D), lambda e, j: (e, j, 0)),
            pl.BlockSpec((1, 1, _D), lambda e, j: (e, 0, 0)),
        ],
        out_specs=pl.BlockSpec((_T, _D), lambda e, j: (0, 0)),
        out_shape=jax.ShapeDtypeStruct((_T, _D), jnp.float32),
        scratch_shapes=[pltpu.VMEM((_T, _E), jnp.float32)],
        compiler_params=pltpu.CompilerParams(
            dimension_semantics=("arbitrary", "arbitrary"),
        ),
    )(x, Wg, bg.reshape(1, _E), W1, b1.reshape(_E, 1, _H), W2,
      b2.reshape(_E, 1, _D))
    return out
